# baseline (speedup 1.0000x reference)
import functools

import jax
import jax.numpy as jnp
from jax import lax
from jax.experimental import pallas as pl
from jax.experimental.pallas import tpu as pltpu
from jax.experimental.pallas import tpu_sc as plsc

_TOP_K = 8
_N_EXP = 64
_ROW_TILE = 1024
_N_CHUNKS = 4


def _matmul_body(x_ref, w_ref, b_ref, o_ref):
    o_ref[...] = jnp.dot(x_ref[...], w_ref[...],
                         preferred_element_type=jnp.float32) + b_ref[...]


def _tc_logits(x, W, b2):
    n_tok, d_model = x.shape
    n_exp = W.shape[1]
    grid = (n_tok // _ROW_TILE,)
    return pl.pallas_call(
        _matmul_body,
        grid=grid,
        in_specs=[
            pl.BlockSpec((_ROW_TILE, d_model), lambda i: (i, 0)),
            pl.BlockSpec((d_model, n_exp), lambda i: (0, 0)),
            pl.BlockSpec((1, n_exp), lambda i: (0, 0)),
        ],
        out_specs=pl.BlockSpec((_ROW_TILE, n_exp), lambda i: (i, 0)),
        out_shape=jax.ShapeDtypeStruct((n_tok, n_exp), jnp.float32),
        compiler_params=pltpu.CompilerParams(
            dimension_semantics=("arbitrary",),
        ),
    )(x, W, b2)


def _make_sc_gate(n_rows):
    info = plsc.get_sparse_core_info()
    n_workers = info.num_cores * info.num_subcores
    rows_per_worker = n_rows // n_workers
    n_groups = rows_per_worker // 16
    grp_words = 16 * _N_EXP
    mesh = plsc.VectorSubcoreMesh(core_axis_name="c", subcore_axis_name="s")

    @functools.partial(
        pl.kernel,
        mesh=mesh,
        out_type=jax.ShapeDtypeStruct((n_rows * _N_EXP,), jnp.float32),
        scratch_types=[
            pltpu.VMEM((grp_words,), jnp.float32),
            pltpu.VMEM((grp_words,), jnp.float32),
        ],
        compiler_params=pltpu.CompilerParams(needs_layout_passes=False),
    )
    def sc_gate(logits_hbm, out_hbm, buf_in, buf_out):
        wid = lax.axis_index("s") * info.num_cores + lax.axis_index("c")
        word0 = wid * rows_per_worker * _N_EXP
        rbase = lax.iota(jnp.int32, 16) * _N_EXP
        neg_inf = jnp.float32(-jnp.inf)
        k_f = jnp.float32(_TOP_K)

        def group(g, _):
            base = word0 + g * grp_words
            pltpu.sync_copy(logits_hbm.at[pl.ds(base, grp_words)], buf_in)
            top = [jnp.full((16,), neg_inf, jnp.float32)
                   for _ in range(_TOP_K)]
            for e in range(_N_EXP):
                v = plsc.load_gather(buf_in, [rbase + e])
                for j in range(_TOP_K):
                    hi = jnp.maximum(top[j], v)
                    v = jnp.minimum(top[j], v)
                    top[j] = hi
            t = top[_TOP_K - 1]
            mx = jnp.maximum(top[0], 0.0)
            n_gt = jnp.zeros((16,), jnp.float32)
            for e in range(_N_EXP):
                v = plsc.load_gather(buf_in, [rbase + e])
                n_gt = n_gt + jnp.where(v > t, 1.0, 0.0)
            need = k_f - n_gt
            e_zero = jnp.exp(-mx)
            s = jnp.zeros((16,), jnp.float32)
            eqc = jnp.zeros((16,), jnp.float32)
            for e in range(_N_EXP):
                v = plsc.load_gather(buf_in, [rbase + e])
                eq = v == t
                take = jnp.logical_and(eq, eqc < need)
                eqc = eqc + jnp.where(eq, 1.0, 0.0)
                keep = jnp.logical_or(v > t, take)
                num = jnp.where(keep, jnp.exp(v - mx), e_zero)
                s = s + num
                plsc.store_scatter(buf_out, [rbase + e], num)
            inv = 1.0 / s
            for e in range(_N_EXP):
                v = plsc.load_gather(buf_out, [rbase + e])
                plsc.store_scatter(buf_out, [rbase + e], v * inv)
            pltpu.sync_copy(buf_out, out_hbm.at[pl.ds(base, grp_words)])
            return ()

        lax.fori_loop(0, n_groups, group, ())

    return sc_gate


@jax.jit
def kernel(x, W, b):
    n_tok = x.shape[0]
    n_exp = W.shape[1]
    b2 = b.reshape(1, n_exp)
    chunk_rows = n_tok // _N_CHUNKS
    sc_gate = _make_sc_gate(chunk_rows)
    outs = []
    for c in range(_N_CHUNKS):
        xc = lax.slice_in_dim(x, c * chunk_rows, (c + 1) * chunk_rows, axis=0)
        logits = _tc_logits(xc, W, b2)
        out_flat = sc_gate(logits.reshape(chunk_rows * n_exp))
        outs.append(out_flat.reshape(chunk_rows, n_exp))
    return jnp.concatenate(outs, axis=0)


# speedup vs baseline: 1.0685x; 1.0685x over previous
import functools

import jax
import jax.numpy as jnp
from jax import lax
from jax.experimental import pallas as pl
from jax.experimental.pallas import tpu as pltpu
from jax.experimental.pallas import tpu_sc as plsc

_TOP_K = 8
_N_EXP = 64
_ROW_TILE = 1024
_N_CHUNKS = 4


def _matmul_body(x_ref, w_ref, b_ref, o_ref):
    o_ref[...] = jnp.dot(x_ref[...], w_ref[...],
                         preferred_element_type=jnp.float32) + b_ref[...]


def _tc_logits(x, W, b2):
    n_tok, d_model = x.shape
    n_exp = W.shape[1]
    grid = (n_tok // _ROW_TILE,)
    return pl.pallas_call(
        _matmul_body,
        grid=grid,
        in_specs=[
            pl.BlockSpec((_ROW_TILE, d_model), lambda i: (i, 0)),
            pl.BlockSpec((d_model, n_exp), lambda i: (0, 0)),
            pl.BlockSpec((1, n_exp), lambda i: (0, 0)),
        ],
        out_specs=pl.BlockSpec((_ROW_TILE, n_exp), lambda i: (i, 0)),
        out_shape=jax.ShapeDtypeStruct((n_tok, n_exp), jnp.float32),
        compiler_params=pltpu.CompilerParams(
            dimension_semantics=("arbitrary",),
        ),
    )(x, W, b2)


def _make_sc_gate(n_rows):
    info = plsc.get_sparse_core_info()
    n_workers = info.num_cores * info.num_subcores
    rows_per_worker = n_rows // n_workers
    n_groups = rows_per_worker // 16
    grp_words = 16 * _N_EXP
    mesh = plsc.VectorSubcoreMesh(core_axis_name="c", subcore_axis_name="s")

    @functools.partial(
        pl.kernel,
        mesh=mesh,
        out_type=jax.ShapeDtypeStruct((n_rows * _N_EXP,), jnp.float32),
        scratch_types=[
            pltpu.VMEM((rows_per_worker * _N_EXP,), jnp.float32),
            pltpu.VMEM((rows_per_worker * _N_EXP,), jnp.float32),
        ],
        compiler_params=pltpu.CompilerParams(needs_layout_passes=False),
    )
    def sc_gate(logits_hbm, out_hbm, buf_in, buf_out):
        wid = lax.axis_index("s") * info.num_cores + lax.axis_index("c")
        word0 = wid * rows_per_worker * _N_EXP
        rbase = lax.iota(jnp.int32, 16) * _N_EXP
        neg_inf = jnp.float32(-jnp.inf)
        pltpu.sync_copy(logits_hbm.at[pl.ds(word0, rows_per_worker * _N_EXP)],
                        buf_in)

        def group(g, _):
            gbase = g * grp_words + rbase
            top = [jnp.full((16,), neg_inf, jnp.float32)
                   for _ in range(_TOP_K)]
            for e in range(_N_EXP):
                v = plsc.load_gather(buf_in, [gbase + e])
                for j in range(_TOP_K):
                    hi = jnp.maximum(top[j], v)
                    v = jnp.minimum(top[j], v)
                    top[j] = hi
            t = top[_TOP_K - 1]
            mx = jnp.maximum(top[0], 0.0)
            need = jnp.zeros((16,), jnp.float32)
            for j in range(_TOP_K):
                need = need + jnp.where(top[j] == t, 1.0, 0.0)
            e_zero = jnp.exp(-mx)
            s = jnp.zeros((16,), jnp.float32)
            eqc = jnp.zeros((16,), jnp.float32)
            for e in range(_N_EXP):
                v = plsc.load_gather(buf_in, [gbase + e])
                eq = v == t
                take = jnp.logical_and(eq, eqc < need)
                eqc = eqc + jnp.where(eq, 1.0, 0.0)
                keep = jnp.logical_or(v > t, take)
                num = jnp.where(keep, jnp.exp(v - mx), e_zero)
                s = s + num
                plsc.store_scatter(buf_out, [gbase + e], num)
            inv = 1.0 / s
            for e in range(_N_EXP):
                v = plsc.load_gather(buf_out, [gbase + e])
                plsc.store_scatter(buf_out, [gbase + e], v * inv)
            return ()

        lax.fori_loop(0, n_groups, group, ())
        pltpu.sync_copy(buf_out,
                        out_hbm.at[pl.ds(word0, rows_per_worker * _N_EXP)])

    return sc_gate


@jax.jit
def kernel(x, W, b):
    n_tok = x.shape[0]
    n_exp = W.shape[1]
    b2 = b.reshape(1, n_exp)
    chunk_rows = n_tok // _N_CHUNKS
    sc_gate = _make_sc_gate(chunk_rows)
    outs = []
    for c in range(_N_CHUNKS):
        xc = lax.slice_in_dim(x, c * chunk_rows, (c + 1) * chunk_rows, axis=0)
        logits = _tc_logits(xc, W, b2)
        out_flat = sc_gate(logits.reshape(chunk_rows * n_exp))
        outs.append(out_flat.reshape(chunk_rows, n_exp))
    return jnp.concatenate(outs, axis=0)


# speedup vs baseline: 4.0224x; 3.7645x over previous
import functools

import jax
import jax.numpy as jnp
from jax.experimental import pallas as pl
from jax.experimental.pallas import tpu as pltpu

_TOP_K = 8
_ROW_TILE = 1024
_CHUNK = 256


def _gate_rows(logits):
    neg_inf = jnp.float32(-jnp.inf)
    k_f = jnp.float32(_TOP_K)
    n_exp = logits.shape[-1]
    cur = logits
    cnt = jnp.zeros(logits.shape[:-1] + (1,), jnp.float32)
    t = jnp.full(logits.shape[:-1] + (1,), neg_inf)
    row_max = None
    for it in range(_TOP_K):
        m = jnp.max(cur, axis=-1, keepdims=True)
        if it == 0:
            row_max = m
            t = m
        else:
            t = jnp.where(cnt < k_f, m, t)
        if it + 1 < _TOP_K:
            eq = cur == m
            cnt = cnt + jnp.sum(jnp.where(eq, 1.0, 0.0),
                                axis=-1, keepdims=True)
            cur = jnp.where(eq, neg_inf, cur)
    gt = logits > t
    eqt = jnp.where(logits == t, 1.0, 0.0)
    n_gt = jnp.sum(jnp.where(gt, 1.0, 0.0), axis=-1, keepdims=True)
    ri = jax.lax.broadcasted_iota(jnp.int32, (n_exp, n_exp), 0)
    ci = jax.lax.broadcasted_iota(jnp.int32, (n_exp, n_exp), 1)
    tri = jnp.where(ri <= ci, 1.0, 0.0)
    rank_eq = jnp.dot(eqt, tri, preferred_element_type=jnp.float32)
    keep = jnp.logical_or(gt, (eqt > 0.0) & (rank_eq <= k_f - n_gt))
    masked = jnp.where(keep, logits, 0.0)
    mx = jnp.maximum(row_max, 0.0)
    e = jnp.exp(masked - mx)
    return e / jnp.sum(e, axis=-1, keepdims=True)


def _matmul_chunk(x_ref, w_ref, b_ref, c):
    rows = pl.ds(c * _CHUNK, _CHUNK)
    return jnp.dot(x_ref[rows, :], w_ref[...],
                   preferred_element_type=jnp.float32) + b_ref[...]


def _gating_body(x_ref, w_ref, b_ref, o_ref):
    n_chunks = _ROW_TILE // _CHUNK
    logits = _matmul_chunk(x_ref, w_ref, b_ref, 0)
    for c in range(n_chunks):
        cur_logits = logits
        if c + 1 < n_chunks:
            logits = _matmul_chunk(x_ref, w_ref, b_ref, c + 1)
        o_ref[pl.ds(c * _CHUNK, _CHUNK), :] = _gate_rows(cur_logits)


@jax.jit
def kernel(x, W, b):
    n_tok, d_model = x.shape
    n_exp = W.shape[1]
    b2 = b.reshape(1, n_exp)
    grid = (n_tok // _ROW_TILE,)
    return pl.pallas_call(
        _gating_body,
        grid=grid,
        in_specs=[
            pl.BlockSpec((_ROW_TILE, d_model), lambda i: (i, 0)),
            pl.BlockSpec((d_model, n_exp), lambda i: (0, 0)),
            pl.BlockSpec((1, n_exp), lambda i: (0, 0)),
        ],
        out_specs=pl.BlockSpec((_ROW_TILE, n_exp), lambda i: (i, 0)),
        out_shape=jax.ShapeDtypeStruct((n_tok, n_exp), jnp.float32),
        compiler_params=pltpu.CompilerParams(
            dimension_semantics=("arbitrary",),
        ),
    )(x, W, b2)
